# baseline (speedup 1.0000x reference)
import functools

import jax
import jax.numpy as jnp
from jax import lax
from jax.experimental import pallas as pl
from jax.experimental.pallas import tpu as pltpu
import jax.experimental.pallas.tpu_sc as plsc

N = 10000
E = 160000
G = 128
D_IN = 256
D_H = 512
D_OUT = 16

NC = 2
NS = 16
NW = NC * NS
L = 16

CW = 128
NCHUNK = D_H // CW
N_PAD = 10240
STRIPE = N_PAD // NS
EB = 64
NB = 80
E_PAD = NW * NB * EB
NB0 = 160
NBUF = 4
NBQ = NB0 // 4
E_ROWS = E_PAD // EB
E_ROWS_ARR = E_ROWS
TILE_N = 1280
GRID_N = N_PAD // TILE_N


def _deg_body(dst_hbm, out_hbm, idx_buf, ones_buf, zstripe, acc_sh):
    core = lax.axis_index("c")
    sid = lax.axis_index("s")
    wid = sid * NC + core
    one16 = jnp.ones((L,), jnp.float32)
    zero16 = jnp.zeros((L,), jnp.float32)

    def fill_ones(t, _):
        ones_buf[pl.ds(t * L, L)] = one16
        return 0
    lax.fori_loop(0, EB // L, fill_ones, 0)

    def fill_z(t, _):
        zstripe[pl.ds(t * L, L)] = zero16
        return 0
    lax.fori_loop(0, STRIPE // L, fill_z, 0)

    pltpu.sync_copy(zstripe, acc_sh.at[pl.ds(sid * STRIPE, STRIPE)])
    pltpu.sync_copy(dst_hbm.at[pl.ds(wid * NB, NB), :], idx_buf)
    plsc.subcore_barrier()

    def add_batch(j, _):
        pltpu.sync_copy(ones_buf, acc_sh.at[idx_buf.at[j]], add=True)
        return 0
    lax.fori_loop(0, NB, add_batch, 0)
    plsc.subcore_barrier()

    pltpu.sync_copy(acc_sh.at[pl.ds(sid * STRIPE, STRIPE)],
                    out_hbm.at[pl.ds(core * N_PAD + sid * STRIPE, STRIPE)])


@functools.cache
def _deg_call():
    mesh = plsc.VectorSubcoreMesh(core_axis_name="c", subcore_axis_name="s",
                                  num_cores=NC, num_subcores=NS)
    return pl.kernel(
        _deg_body,
        out_type=jax.ShapeDtypeStruct((NC * N_PAD,), jnp.float32),
        mesh=mesh,
        scratch_types=[
            pltpu.VMEM((NB, EB), jnp.int32),
            pltpu.VMEM((EB,), jnp.float32),
            pltpu.VMEM((STRIPE,), jnp.float32),
            pltpu.VMEM_SHARED((N_PAD,), jnp.float32),
        ],
    )


def _spmm_body(mp_hbm, src_hbm, dst_hbm, out_hbm,
               src_buf, dst_buf, rows_a, rows_b, rows_c, rows_d,
               sem_a, sem_b, sem_c, sem_d, acc_sh):
    core = lax.axis_index("c")
    sid = lax.axis_index("s")
    zero16 = jnp.zeros((L,), jnp.float32)
    bufs = ((rows_a, sem_a), (rows_b, sem_b), (rows_c, sem_c), (rows_d, sem_d))

    active = core == 0
    base = sid * NB0
    nsteps = jnp.where(active, NBQ // NBUF, 0)

    for c in range(NCHUNK):
        def fill_z(t, _):
            r = t // (CW // L)
            k = (t % (CW // L)) * L
            rows_a[r, pl.ds(k, L)] = zero16
            return 0
        lax.fori_loop(0, EB * CW // L, fill_z, 0)
        for r in range(STRIPE // EB):
            pltpu.sync_copy(rows_a, acc_sh.at[pl.ds(sid * STRIPE + r * EB, EB), :])
        plsc.subcore_barrier()

        table = mp_hbm.at[c]
        for q in range(4):
            qbase = base + q * NBQ

            @pl.when(active)
            def _():
                pltpu.sync_copy(src_hbm.at[pl.ds(qbase, NBQ), :], src_buf)
                pltpu.sync_copy(dst_hbm.at[pl.ds(qbase, NBQ), :], dst_buf)
                for b, (rbuf, sem) in enumerate(bufs):
                    pltpu.async_copy(table.at[src_buf.at[b]], rbuf, sem)

            def step(t, _):
                for b, (rbuf, sem) in enumerate(bufs):
                    j = NBUF * t + b
                    pltpu.make_async_copy(table.at[src_buf.at[j]], rbuf,
                                          sem).wait()
                    pltpu.sync_copy(rbuf, acc_sh.at[dst_buf.at[j]], add=True)
                    jj = j + NBUF

                    @pl.when(jj < NBQ)
                    def _():
                        pltpu.async_copy(table.at[src_buf.at[jj]], rbuf, sem)
                return 0
            lax.fori_loop(0, nsteps, step, 0)
        plsc.subcore_barrier()

        @pl.when(active)
        def _():
            pltpu.sync_copy(
                acc_sh.at[pl.ds(sid * STRIPE, STRIPE), :],
                out_hbm.at[c].at[pl.ds(sid * STRIPE, STRIPE), :])
        plsc.subcore_barrier()


@functools.cache
def _spmm_call():
    mesh = plsc.VectorSubcoreMesh(core_axis_name="c", subcore_axis_name="s",
                                  num_cores=NC, num_subcores=NS)
    return pl.kernel(
        _spmm_body,
        out_type=jax.ShapeDtypeStruct((NCHUNK, N_PAD, CW), jnp.float32),
        mesh=mesh,
        scratch_types=[
            pltpu.VMEM((NBQ, EB), jnp.int32),
            pltpu.VMEM((NBQ, EB), jnp.int32),
            pltpu.VMEM((EB, CW), jnp.float32),
            pltpu.VMEM((EB, CW), jnp.float32),
            pltpu.VMEM((EB, CW), jnp.float32),
            pltpu.VMEM((EB, CW), jnp.float32),
            pltpu.SemaphoreType.DMA,
            pltpu.SemaphoreType.DMA,
            pltpu.SemaphoreType.DMA,
            pltpu.SemaphoreType.DMA,
            pltpu.VMEM_SHARED((N_PAD, CW), jnp.float32),
        ],
    )


def _mm1_body(x_ref, w_ref, dinv_ref, out_ref):
    m = jnp.dot(x_ref[...], w_ref[...], preferred_element_type=jnp.float32)
    m = m * dinv_ref[...]
    for c in range(NCHUNK):
        out_ref[c] = m[:, c * CW:(c + 1) * CW]


_mm1_call = pl.pallas_call(
    _mm1_body,
    grid=(GRID_N,),
    in_specs=[
        pl.BlockSpec((TILE_N, D_IN), lambda i: (i, 0)),
        pl.BlockSpec((D_IN, D_H), lambda i: (0, 0)),
        pl.BlockSpec((TILE_N, 1), lambda i: (i, 0)),
    ],
    out_specs=pl.BlockSpec((NCHUNK, TILE_N, CW), lambda i: (0, i, 0)),
    out_shape=jax.ShapeDtypeStruct((NCHUNK, N_PAD, CW), jnp.float32),
)


def _mm2_body(s_ref, m_ref, dinv_ref, b_ref, w_ref, out_ref):
    dinv = dinv_ref[...]
    acc = jnp.zeros((TILE_N, D_H), jnp.float32)
    for c in range(NCHUNK):
        mc = m_ref[c]
        sc = s_ref[c] + mc
        hc = jnp.maximum(sc * dinv + b_ref[:, c * CW:(c + 1) * CW], 0.0)
        acc = acc + jnp.dot(hc, w_ref[c * CW:(c + 1) * CW, :],
                            preferred_element_type=jnp.float32)
    acc = acc * dinv
    for c in range(NCHUNK):
        out_ref[c] = acc[:, c * CW:(c + 1) * CW]


_mm2_call = pl.pallas_call(
    _mm2_body,
    grid=(GRID_N,),
    in_specs=[
        pl.BlockSpec((NCHUNK, TILE_N, CW), lambda i: (0, i, 0)),
        pl.BlockSpec((NCHUNK, TILE_N, CW), lambda i: (0, i, 0)),
        pl.BlockSpec((TILE_N, 1), lambda i: (i, 0)),
        pl.BlockSpec((1, D_H), lambda i: (0, 0)),
        pl.BlockSpec((D_H, D_H), lambda i: (0, 0)),
    ],
    out_specs=pl.BlockSpec((NCHUNK, TILE_N, CW), lambda i: (0, i, 0)),
    out_shape=jax.ShapeDtypeStruct((NCHUNK, N_PAD, CW), jnp.float32),
)


def _fin_body(s_ref, m_ref, dinv_ref, b_ref, batch_ref, wfc_ref, bfc_ref,
              out_ref, pooled_acc, cnt_acc):
    i = pl.program_id(0)

    @pl.when(i == 0)
    def _():
        pooled_acc[...] = jnp.zeros_like(pooled_acc)
        cnt_acc[...] = jnp.zeros_like(cnt_acc)

    dinv = dinv_ref[...]
    bt = batch_ref[...]
    gi = lax.broadcasted_iota(jnp.int32, (G, TILE_N), 0)
    oh = jnp.where(gi == bt, 1.0, 0.0)
    cnt_acc[...] += jnp.sum(oh, axis=1, keepdims=True)
    for c in range(NCHUNK):
        mc = m_ref[c]
        sc = s_ref[c] + mc
        hc = jnp.maximum(sc * dinv + b_ref[:, c * CW:(c + 1) * CW], 0.0)
        pooled_acc[c] += jnp.dot(oh, hc, preferred_element_type=jnp.float32)

    @pl.when(i == GRID_N - 1)
    def _():
        cnt = jnp.maximum(cnt_acc[...], 1.0)
        logits = jnp.zeros((G, D_OUT), jnp.float32)
        for c in range(NCHUNK):
            logits = logits + jnp.dot(pooled_acc[c] / cnt,
                                      wfc_ref[c * CW:(c + 1) * CW, :],
                                      preferred_element_type=jnp.float32)
        logits = logits + bfc_ref[...]
        mx = jnp.max(logits, axis=1, keepdims=True)
        lse = jnp.log(jnp.sum(jnp.exp(logits - mx), axis=1, keepdims=True)) + mx
        out_ref[...] = logits - lse


_fin_call = pl.pallas_call(
    _fin_body,
    grid=(GRID_N,),
    in_specs=[
        pl.BlockSpec((NCHUNK, TILE_N, CW), lambda i: (0, i, 0)),
        pl.BlockSpec((NCHUNK, TILE_N, CW), lambda i: (0, i, 0)),
        pl.BlockSpec((TILE_N, 1), lambda i: (i, 0)),
        pl.BlockSpec((1, D_H), lambda i: (0, 0)),
        pl.BlockSpec((1, TILE_N), lambda i: (0, i)),
        pl.BlockSpec((D_H, D_OUT), lambda i: (0, 0)),
        pl.BlockSpec((1, D_OUT), lambda i: (0, 0)),
    ],
    out_specs=pl.BlockSpec((G, D_OUT), lambda i: (0, 0)),
    out_shape=jax.ShapeDtypeStruct((G, D_OUT), jnp.float32),
    scratch_shapes=[
        pltpu.VMEM((NCHUNK, G, CW), jnp.float32),
        pltpu.VMEM((G, 1), jnp.float32),
    ],
)


def kernel(x, edge_index, batch, W1, b1, W2, b2, Wfc, bfc):
    src = edge_index[0]
    dst = edge_index[1]
    pe = E_ROWS_ARR * EB - E
    srcr = jnp.concatenate([src, jnp.zeros((pe,), jnp.int32)]).reshape(E_ROWS_ARR, EB)
    dstr = jnp.concatenate([dst, jnp.full((pe,), N, jnp.int32)]).reshape(E_ROWS_ARR, EB)
    xp = jnp.pad(x, ((0, N_PAD - N), (0, 0)))
    batch_p = jnp.concatenate([batch, jnp.full((N_PAD - N,), G, jnp.int32)]).reshape(1, N_PAD)

    degp = _deg_call()(dstr)
    deg = degp.reshape(NC, N_PAD).sum(axis=0) + 1.0
    dinv = lax.rsqrt(deg).reshape(N_PAD, 1)

    spmm = _spmm_call()
    m1 = _mm1_call(xp, W1, dinv)
    s1 = spmm(m1, srcr, dstr)
    m2 = _mm2_call(s1, m1, dinv, b1.reshape(1, D_H), W2)
    s2 = spmm(m2, srcr, dstr)
    return _fin_call(s2, m2, dinv, b2.reshape(1, D_H), batch_p, Wfc,
                     bfc.reshape(1, D_OUT))


# speedup vs baseline: 1.3352x; 1.3352x over previous
import functools

import jax
import jax.numpy as jnp
from jax import lax
from jax.experimental import pallas as pl
from jax.experimental.pallas import tpu as pltpu
import jax.experimental.pallas.tpu_sc as plsc

N = 10000
E = 160000
G = 128
D_IN = 256
D_H = 512
D_OUT = 16

NC = 2
NS = 16
NW = NC * NS
L = 16

CW = 128
NCHUNK = D_H // CW
N_PAD = 10240
STRIPE = N_PAD // NS
EB = 64
NB = 80
E_PAD = NW * NB * EB
NB0 = 160
NBUF = 4
NBQ = NB0 // 4
E_ROWS = E_PAD // EB
E_ROWS_ARR = E_ROWS
TILE_N = 1280
GRID_N = N_PAD // TILE_N


def _deg_body(dst_hbm, out_hbm, idx_buf, ones_buf, zstripe, acc_sh):
    core = lax.axis_index("c")
    sid = lax.axis_index("s")
    wid = sid * NC + core
    one16 = jnp.ones((L,), jnp.float32)
    zero16 = jnp.zeros((L,), jnp.float32)

    def fill_ones(t, _):
        ones_buf[pl.ds(t * L, L)] = one16
        return 0
    lax.fori_loop(0, EB // L, fill_ones, 0)

    def fill_z(t, _):
        zstripe[pl.ds(t * L, L)] = zero16
        return 0
    lax.fori_loop(0, STRIPE // L, fill_z, 0)

    pltpu.sync_copy(zstripe, acc_sh.at[pl.ds(sid * STRIPE, STRIPE)])
    pltpu.sync_copy(dst_hbm.at[pl.ds(wid * NB, NB), :], idx_buf)
    plsc.subcore_barrier()

    def add_batch(j, _):
        pltpu.sync_copy(ones_buf, acc_sh.at[idx_buf.at[j]], add=True)
        return 0
    lax.fori_loop(0, NB, add_batch, 0)
    plsc.subcore_barrier()

    pltpu.sync_copy(acc_sh.at[pl.ds(sid * STRIPE, STRIPE)],
                    out_hbm.at[pl.ds(core * N_PAD + sid * STRIPE, STRIPE)])


@functools.cache
def _deg_call():
    mesh = plsc.VectorSubcoreMesh(core_axis_name="c", subcore_axis_name="s",
                                  num_cores=NC, num_subcores=NS)
    return pl.kernel(
        _deg_body,
        out_type=jax.ShapeDtypeStruct((NC * N_PAD,), jnp.float32),
        mesh=mesh,
        scratch_types=[
            pltpu.VMEM((NB, EB), jnp.int32),
            pltpu.VMEM((EB,), jnp.float32),
            pltpu.VMEM((STRIPE,), jnp.float32),
            pltpu.VMEM_SHARED((N_PAD,), jnp.float32),
        ],
    )


def _spmm_body(mp_hbm, src_hbm, dst_hbm, out_hbm,
               src_buf, dst_buf, rows_a, rows_b, rows_c, rows_d,
               sem_a, sem_b, sem_c, sem_d, acc_sh):
    core = lax.axis_index("c")
    sid = lax.axis_index("s")
    zero16 = jnp.zeros((L,), jnp.float32)
    bufs = ((rows_a, sem_a), (rows_b, sem_b), (rows_c, sem_c), (rows_d, sem_d))

    base = sid * NB0

    for c in range(NCHUNK):
        owner = 0 if c < NCHUNK - 1 else 1
        mine = core == owner
        nsteps = jnp.where(mine, NBQ // NBUF, 0)

        def fill_z(t, _):
            r = t // (CW // L)
            k = (t % (CW // L)) * L
            rows_a[r, pl.ds(k, L)] = zero16
            return 0
        lax.fori_loop(0, EB * CW // L, fill_z, 0)

        @pl.when(mine)
        def _():
            for r in range(STRIPE // EB):
                pltpu.sync_copy(rows_a,
                                acc_sh.at[pl.ds(sid * STRIPE + r * EB, EB), :])
        plsc.subcore_barrier()

        table = mp_hbm.at[c]
        for q in range(4):
            qbase = base + q * NBQ

            @pl.when(mine)
            def _():
                pltpu.sync_copy(src_hbm.at[pl.ds(qbase, NBQ), :], src_buf)
                pltpu.sync_copy(dst_hbm.at[pl.ds(qbase, NBQ), :], dst_buf)
                for b, (rbuf, sem) in enumerate(bufs):
                    pltpu.async_copy(table.at[src_buf.at[b]], rbuf, sem)

            def step(t, _):
                for b, (rbuf, sem) in enumerate(bufs):
                    j = NBUF * t + b
                    pltpu.make_async_copy(table.at[src_buf.at[j]], rbuf,
                                          sem).wait()
                    pltpu.sync_copy(rbuf, acc_sh.at[dst_buf.at[j]], add=True)
                    jj = j + NBUF

                    @pl.when(jj < NBQ)
                    def _():
                        pltpu.async_copy(table.at[src_buf.at[jj]], rbuf, sem)
                return 0
            lax.fori_loop(0, nsteps, step, 0)
        plsc.subcore_barrier()

        @pl.when(mine)
        def _():
            pltpu.sync_copy(
                acc_sh.at[pl.ds(sid * STRIPE, STRIPE), :],
                out_hbm.at[c].at[pl.ds(sid * STRIPE, STRIPE), :])
        plsc.subcore_barrier()


@functools.cache
def _spmm_call():
    mesh = plsc.VectorSubcoreMesh(core_axis_name="c", subcore_axis_name="s",
                                  num_cores=NC, num_subcores=NS)
    return pl.kernel(
        _spmm_body,
        out_type=jax.ShapeDtypeStruct((NCHUNK, N_PAD, CW), jnp.float32),
        mesh=mesh,
        scratch_types=[
            pltpu.VMEM((NBQ, EB), jnp.int32),
            pltpu.VMEM((NBQ, EB), jnp.int32),
            pltpu.VMEM((EB, CW), jnp.float32),
            pltpu.VMEM((EB, CW), jnp.float32),
            pltpu.VMEM((EB, CW), jnp.float32),
            pltpu.VMEM((EB, CW), jnp.float32),
            pltpu.SemaphoreType.DMA,
            pltpu.SemaphoreType.DMA,
            pltpu.SemaphoreType.DMA,
            pltpu.SemaphoreType.DMA,
            pltpu.VMEM_SHARED((N_PAD, CW), jnp.float32),
        ],
    )


def _mm1_body(x_ref, w_ref, dinv_ref, out_ref):
    m = jnp.dot(x_ref[...], w_ref[...], preferred_element_type=jnp.float32)
    m = m * dinv_ref[...]
    for c in range(NCHUNK):
        out_ref[c] = m[:, c * CW:(c + 1) * CW]


_mm1_call = pl.pallas_call(
    _mm1_body,
    grid=(GRID_N,),
    in_specs=[
        pl.BlockSpec((TILE_N, D_IN), lambda i: (i, 0)),
        pl.BlockSpec((D_IN, D_H), lambda i: (0, 0)),
        pl.BlockSpec((TILE_N, 1), lambda i: (i, 0)),
    ],
    out_specs=pl.BlockSpec((NCHUNK, TILE_N, CW), lambda i: (0, i, 0)),
    out_shape=jax.ShapeDtypeStruct((NCHUNK, N_PAD, CW), jnp.float32),
)


def _mm2_body(s_ref, m_ref, dinv_ref, b_ref, w_ref, out_ref):
    dinv = dinv_ref[...]
    acc = jnp.zeros((TILE_N, D_H), jnp.float32)
    for c in range(NCHUNK):
        mc = m_ref[c]
        sc = s_ref[c] + mc
        hc = jnp.maximum(sc * dinv + b_ref[:, c * CW:(c + 1) * CW], 0.0)
        acc = acc + jnp.dot(hc, w_ref[c * CW:(c + 1) * CW, :],
                            preferred_element_type=jnp.float32)
    acc = acc * dinv
    for c in range(NCHUNK):
        out_ref[c] = acc[:, c * CW:(c + 1) * CW]


_mm2_call = pl.pallas_call(
    _mm2_body,
    grid=(GRID_N,),
    in_specs=[
        pl.BlockSpec((NCHUNK, TILE_N, CW), lambda i: (0, i, 0)),
        pl.BlockSpec((NCHUNK, TILE_N, CW), lambda i: (0, i, 0)),
        pl.BlockSpec((TILE_N, 1), lambda i: (i, 0)),
        pl.BlockSpec((1, D_H), lambda i: (0, 0)),
        pl.BlockSpec((D_H, D_H), lambda i: (0, 0)),
    ],
    out_specs=pl.BlockSpec((NCHUNK, TILE_N, CW), lambda i: (0, i, 0)),
    out_shape=jax.ShapeDtypeStruct((NCHUNK, N_PAD, CW), jnp.float32),
)


def _fin_body(s_ref, m_ref, dinv_ref, b_ref, batch_ref, wfc_ref, bfc_ref,
              out_ref, pooled_acc, cnt_acc):
    i = pl.program_id(0)

    @pl.when(i == 0)
    def _():
        pooled_acc[...] = jnp.zeros_like(pooled_acc)
        cnt_acc[...] = jnp.zeros_like(cnt_acc)

    dinv = dinv_ref[...]
    bt = batch_ref[...]
    gi = lax.broadcasted_iota(jnp.int32, (G, TILE_N), 0)
    oh = jnp.where(gi == bt, 1.0, 0.0)
    cnt_acc[...] += jnp.sum(oh, axis=1, keepdims=True)
    for c in range(NCHUNK):
        mc = m_ref[c]
        sc = s_ref[c] + mc
        hc = jnp.maximum(sc * dinv + b_ref[:, c * CW:(c + 1) * CW], 0.0)
        pooled_acc[c] += jnp.dot(oh, hc, preferred_element_type=jnp.float32)

    @pl.when(i == GRID_N - 1)
    def _():
        cnt = jnp.maximum(cnt_acc[...], 1.0)
        logits = jnp.zeros((G, D_OUT), jnp.float32)
        for c in range(NCHUNK):
            logits = logits + jnp.dot(pooled_acc[c] / cnt,
                                      wfc_ref[c * CW:(c + 1) * CW, :],
                                      preferred_element_type=jnp.float32)
        logits = logits + bfc_ref[...]
        mx = jnp.max(logits, axis=1, keepdims=True)
        lse = jnp.log(jnp.sum(jnp.exp(logits - mx), axis=1, keepdims=True)) + mx
        out_ref[...] = logits - lse


_fin_call = pl.pallas_call(
    _fin_body,
    grid=(GRID_N,),
    in_specs=[
        pl.BlockSpec((NCHUNK, TILE_N, CW), lambda i: (0, i, 0)),
        pl.BlockSpec((NCHUNK, TILE_N, CW), lambda i: (0, i, 0)),
        pl.BlockSpec((TILE_N, 1), lambda i: (i, 0)),
        pl.BlockSpec((1, D_H), lambda i: (0, 0)),
        pl.BlockSpec((1, TILE_N), lambda i: (0, i)),
        pl.BlockSpec((D_H, D_OUT), lambda i: (0, 0)),
        pl.BlockSpec((1, D_OUT), lambda i: (0, 0)),
    ],
    out_specs=pl.BlockSpec((G, D_OUT), lambda i: (0, 0)),
    out_shape=jax.ShapeDtypeStruct((G, D_OUT), jnp.float32),
    scratch_shapes=[
        pltpu.VMEM((NCHUNK, G, CW), jnp.float32),
        pltpu.VMEM((G, 1), jnp.float32),
    ],
)


def kernel(x, edge_index, batch, W1, b1, W2, b2, Wfc, bfc):
    src = edge_index[0]
    dst = edge_index[1]
    pe = E_ROWS_ARR * EB - E
    srcr = jnp.concatenate([src, jnp.zeros((pe,), jnp.int32)]).reshape(E_ROWS_ARR, EB)
    dstr = jnp.concatenate([dst, jnp.full((pe,), N, jnp.int32)]).reshape(E_ROWS_ARR, EB)
    xp = jnp.pad(x, ((0, N_PAD - N), (0, 0)))
    batch_p = jnp.concatenate([batch, jnp.full((N_PAD - N,), G, jnp.int32)]).reshape(1, N_PAD)

    degp = _deg_call()(dstr)
    deg = degp.reshape(NC, N_PAD).sum(axis=0) + 1.0
    dinv = lax.rsqrt(deg).reshape(N_PAD, 1)

    spmm = _spmm_call()
    m1 = _mm1_call(xp, W1, dinv)
    s1 = spmm(m1, srcr, dstr)
    m2 = _mm2_call(s1, m1, dinv, b1.reshape(1, D_H), W2)
    s2 = spmm(m2, srcr, dstr)
    return _fin_call(s2, m2, dinv, b2.reshape(1, D_H), batch_p, Wfc,
                     bfc.reshape(1, D_OUT))


# speedup vs baseline: 1.8192x; 1.3625x over previous
import functools

import jax
import jax.numpy as jnp
from jax import lax
from jax.experimental import pallas as pl
from jax.experimental.pallas import tpu as pltpu
import jax.experimental.pallas.tpu_sc as plsc

N = 10000
E = 160000
G = 128
D_IN = 256
D_H = 512
D_OUT = 16

NC = 2
NS = 16
NW = NC * NS
L = 16

CW = 128
NCHUNK = D_H // CW
N_PAD = 10240
STRIPE = N_PAD // NS
EB = 64
NB = 80
E_PAD = NW * NB * EB
NB0 = 160
NBUF = 4
NBQ = NB0 // 4
E_ROWS = E_PAD // EB
E_ROWS_ARR = E_ROWS
TILE_N = 1280
GRID_N = N_PAD // TILE_N


def _deg_body(dst_hbm, out_hbm, idx_buf, ones_buf, zstripe, acc_sh):
    core = lax.axis_index("c")
    sid = lax.axis_index("s")
    wid = sid * NC + core
    one16 = jnp.ones((L,), jnp.float32)
    zero16 = jnp.zeros((L,), jnp.float32)

    def fill_ones(t, _):
        ones_buf[pl.ds(t * L, L)] = one16
        return 0
    lax.fori_loop(0, EB // L, fill_ones, 0)

    def fill_z(t, _):
        zstripe[pl.ds(t * L, L)] = zero16
        return 0
    lax.fori_loop(0, STRIPE // L, fill_z, 0)

    pltpu.sync_copy(zstripe, acc_sh.at[pl.ds(sid * STRIPE, STRIPE)])
    pltpu.sync_copy(dst_hbm.at[pl.ds(wid * NB, NB), :], idx_buf)
    plsc.subcore_barrier()

    def add_batch(j, _):
        pltpu.sync_copy(ones_buf, acc_sh.at[idx_buf.at[j]], add=True)
        return 0
    lax.fori_loop(0, NB, add_batch, 0)
    plsc.subcore_barrier()

    pltpu.sync_copy(acc_sh.at[pl.ds(sid * STRIPE, STRIPE)],
                    out_hbm.at[pl.ds(core * N_PAD + sid * STRIPE, STRIPE)])


@functools.cache
def _deg_call():
    mesh = plsc.VectorSubcoreMesh(core_axis_name="c", subcore_axis_name="s",
                                  num_cores=NC, num_subcores=NS)
    return pl.kernel(
        _deg_body,
        out_type=jax.ShapeDtypeStruct((NC * N_PAD,), jnp.float32),
        mesh=mesh,
        scratch_types=[
            pltpu.VMEM((NB, EB), jnp.int32),
            pltpu.VMEM((EB,), jnp.float32),
            pltpu.VMEM((STRIPE,), jnp.float32),
            pltpu.VMEM_SHARED((N_PAD,), jnp.float32),
        ],
    )


def _spmm_body(mp_hbm, src_hbm, dst_hbm, out_hbm,
               src_buf, dst_buf, rows_a, rows_b, rows_c, rows_d,
               sem_a, sem_b, sem_c, sem_d, acc_sh):
    core = lax.axis_index("c")
    sid = lax.axis_index("s")
    zero16 = jnp.zeros((L,), jnp.float32)
    bufs = ((rows_a, sem_a), (rows_b, sem_b), (rows_c, sem_c), (rows_d, sem_d))

    base = sid * NB0

    for c in range(NCHUNK):
        owner = 0 if c < NCHUNK // 2 else 1
        mine = core == owner
        nsteps = jnp.where(mine, NBQ // NBUF, 0)

        def fill_z(t, _):
            r = t // (CW // L)
            k = (t % (CW // L)) * L
            rows_a[r, pl.ds(k, L)] = zero16
            return 0
        lax.fori_loop(0, EB * CW // L, fill_z, 0)

        @pl.when(mine)
        def _():
            for r in range(STRIPE // EB):
                pltpu.sync_copy(rows_a,
                                acc_sh.at[pl.ds(sid * STRIPE + r * EB, EB), :])
        plsc.subcore_barrier()

        table = mp_hbm.at[c]
        for q in range(4):
            qbase = base + q * NBQ

            @pl.when(mine)
            def _():
                pltpu.sync_copy(src_hbm.at[pl.ds(qbase, NBQ), :], src_buf)
                pltpu.sync_copy(dst_hbm.at[pl.ds(qbase, NBQ), :], dst_buf)
                for b, (rbuf, sem) in enumerate(bufs):
                    pltpu.async_copy(table.at[src_buf.at[b]], rbuf, sem)

            def step(t, _):
                for b, (rbuf, sem) in enumerate(bufs):
                    j = NBUF * t + b
                    pltpu.make_async_copy(table.at[src_buf.at[j]], rbuf,
                                          sem).wait()
                    pltpu.sync_copy(rbuf, acc_sh.at[dst_buf.at[j]], add=True)
                    jj = j + NBUF

                    @pl.when(jj < NBQ)
                    def _():
                        pltpu.async_copy(table.at[src_buf.at[jj]], rbuf, sem)
                return 0
            lax.fori_loop(0, nsteps, step, 0)
        plsc.subcore_barrier()

        @pl.when(mine)
        def _():
            pltpu.sync_copy(
                acc_sh.at[pl.ds(sid * STRIPE, STRIPE), :],
                out_hbm.at[c].at[pl.ds(sid * STRIPE, STRIPE), :])
        plsc.subcore_barrier()


@functools.cache
def _spmm_call():
    mesh = plsc.VectorSubcoreMesh(core_axis_name="c", subcore_axis_name="s",
                                  num_cores=NC, num_subcores=NS)
    return pl.kernel(
        _spmm_body,
        out_type=jax.ShapeDtypeStruct((NCHUNK, N_PAD, CW), jnp.float32),
        mesh=mesh,
        scratch_types=[
            pltpu.VMEM((NBQ, EB), jnp.int32),
            pltpu.VMEM((NBQ, EB), jnp.int32),
            pltpu.VMEM((EB, CW), jnp.float32),
            pltpu.VMEM((EB, CW), jnp.float32),
            pltpu.VMEM((EB, CW), jnp.float32),
            pltpu.VMEM((EB, CW), jnp.float32),
            pltpu.SemaphoreType.DMA,
            pltpu.SemaphoreType.DMA,
            pltpu.SemaphoreType.DMA,
            pltpu.SemaphoreType.DMA,
            pltpu.VMEM_SHARED((N_PAD, CW), jnp.float32),
        ],
    )


def _mm1_body(x_ref, w_ref, dinv_ref, out_ref):
    m = jnp.dot(x_ref[...], w_ref[...], preferred_element_type=jnp.float32)
    m = m * dinv_ref[...]
    for c in range(NCHUNK):
        out_ref[c] = m[:, c * CW:(c + 1) * CW]


_mm1_call = pl.pallas_call(
    _mm1_body,
    grid=(GRID_N,),
    in_specs=[
        pl.BlockSpec((TILE_N, D_IN), lambda i: (i, 0)),
        pl.BlockSpec((D_IN, D_H), lambda i: (0, 0)),
        pl.BlockSpec((TILE_N, 1), lambda i: (i, 0)),
    ],
    out_specs=pl.BlockSpec((NCHUNK, TILE_N, CW), lambda i: (0, i, 0)),
    out_shape=jax.ShapeDtypeStruct((NCHUNK, N_PAD, CW), jnp.float32),
)


def _mm2_body(s_ref, m_ref, dinv_ref, b_ref, w_ref, out_ref):
    dinv = dinv_ref[...]
    acc = jnp.zeros((TILE_N, D_H), jnp.float32)
    for c in range(NCHUNK):
        mc = m_ref[c]
        sc = s_ref[c] + mc
        hc = jnp.maximum(sc * dinv + b_ref[:, c * CW:(c + 1) * CW], 0.0)
        acc = acc + jnp.dot(hc, w_ref[c * CW:(c + 1) * CW, :],
                            preferred_element_type=jnp.float32)
    acc = acc * dinv
    for c in range(NCHUNK):
        out_ref[c] = acc[:, c * CW:(c + 1) * CW]


_mm2_call = pl.pallas_call(
    _mm2_body,
    grid=(GRID_N,),
    in_specs=[
        pl.BlockSpec((NCHUNK, TILE_N, CW), lambda i: (0, i, 0)),
        pl.BlockSpec((NCHUNK, TILE_N, CW), lambda i: (0, i, 0)),
        pl.BlockSpec((TILE_N, 1), lambda i: (i, 0)),
        pl.BlockSpec((1, D_H), lambda i: (0, 0)),
        pl.BlockSpec((D_H, D_H), lambda i: (0, 0)),
    ],
    out_specs=pl.BlockSpec((NCHUNK, TILE_N, CW), lambda i: (0, i, 0)),
    out_shape=jax.ShapeDtypeStruct((NCHUNK, N_PAD, CW), jnp.float32),
)


def _fin_body(s_ref, m_ref, dinv_ref, b_ref, batch_ref, wfc_ref, bfc_ref,
              out_ref, pooled_acc, cnt_acc):
    i = pl.program_id(0)

    @pl.when(i == 0)
    def _():
        pooled_acc[...] = jnp.zeros_like(pooled_acc)
        cnt_acc[...] = jnp.zeros_like(cnt_acc)

    dinv = dinv_ref[...]
    bt = batch_ref[...]
    gi = lax.broadcasted_iota(jnp.int32, (G, TILE_N), 0)
    oh = jnp.where(gi == bt, 1.0, 0.0)
    cnt_acc[...] += jnp.sum(oh, axis=1, keepdims=True)
    for c in range(NCHUNK):
        mc = m_ref[c]
        sc = s_ref[c] + mc
        hc = jnp.maximum(sc * dinv + b_ref[:, c * CW:(c + 1) * CW], 0.0)
        pooled_acc[c] += jnp.dot(oh, hc, preferred_element_type=jnp.float32)

    @pl.when(i == GRID_N - 1)
    def _():
        cnt = jnp.maximum(cnt_acc[...], 1.0)
        logits = jnp.zeros((G, D_OUT), jnp.float32)
        for c in range(NCHUNK):
            logits = logits + jnp.dot(pooled_acc[c] / cnt,
                                      wfc_ref[c * CW:(c + 1) * CW, :],
                                      preferred_element_type=jnp.float32)
        logits = logits + bfc_ref[...]
        mx = jnp.max(logits, axis=1, keepdims=True)
        lse = jnp.log(jnp.sum(jnp.exp(logits - mx), axis=1, keepdims=True)) + mx
        out_ref[...] = logits - lse


_fin_call = pl.pallas_call(
    _fin_body,
    grid=(GRID_N,),
    in_specs=[
        pl.BlockSpec((NCHUNK, TILE_N, CW), lambda i: (0, i, 0)),
        pl.BlockSpec((NCHUNK, TILE_N, CW), lambda i: (0, i, 0)),
        pl.BlockSpec((TILE_N, 1), lambda i: (i, 0)),
        pl.BlockSpec((1, D_H), lambda i: (0, 0)),
        pl.BlockSpec((1, TILE_N), lambda i: (0, i)),
        pl.BlockSpec((D_H, D_OUT), lambda i: (0, 0)),
        pl.BlockSpec((1, D_OUT), lambda i: (0, 0)),
    ],
    out_specs=pl.BlockSpec((G, D_OUT), lambda i: (0, 0)),
    out_shape=jax.ShapeDtypeStruct((G, D_OUT), jnp.float32),
    scratch_shapes=[
        pltpu.VMEM((NCHUNK, G, CW), jnp.float32),
        pltpu.VMEM((G, 1), jnp.float32),
    ],
)


def kernel(x, edge_index, batch, W1, b1, W2, b2, Wfc, bfc):
    src = edge_index[0]
    dst = edge_index[1]
    pe = E_ROWS_ARR * EB - E
    srcr = jnp.concatenate([src, jnp.zeros((pe,), jnp.int32)]).reshape(E_ROWS_ARR, EB)
    dstr = jnp.concatenate([dst, jnp.full((pe,), N, jnp.int32)]).reshape(E_ROWS_ARR, EB)
    xp = jnp.pad(x, ((0, N_PAD - N), (0, 0)))
    batch_p = jnp.concatenate([batch, jnp.full((N_PAD - N,), G, jnp.int32)]).reshape(1, N_PAD)

    degp = _deg_call()(dstr)
    deg = degp.reshape(NC, N_PAD).sum(axis=0) + 1.0
    dinv = lax.rsqrt(deg).reshape(N_PAD, 1)

    spmm = _spmm_call()
    m1 = _mm1_call(xp, W1, dinv)
    s1 = spmm(m1, srcr, dstr)
    m2 = _mm2_call(s1, m1, dinv, b1.reshape(1, D_H), W2)
    s2 = spmm(m2, srcr, dstr)
    return _fin_call(s2, m2, dinv, b2.reshape(1, D_H), batch_p, Wfc,
                     bfc.reshape(1, D_OUT))
